# Initial kernel scaffold; baseline (speedup 1.0000x reference)
import jax
import jax.numpy as jnp
from jax.experimental import pallas as pl


def kernel(W, edge_weight, edge_index):
    raise NotImplementedError("write your pallas kernel here")


# speedup vs baseline: 4.1357x; 4.1357x over previous
import functools

import jax
import jax.numpy as jnp
from jax import lax
from jax.experimental import pallas as pl
from jax.experimental.pallas import tpu as pltpu
from jax.experimental.pallas import tpu_sc as plsc

N = 10000
D = 128
E = 320000
NC = 2
NS = 16
NW = NC * NS
L = 16
CH = 128
EPW = E // NW
NCHK = -(-EPW // CH)
EPW_P = NCHK * CH
ROWS_PT = 624
TAIL0 = NS * ROWS_PT
TAILR = N - TAIL0

_MESH = plsc.VectorSubcoreMesh(
    core_axis_name="c", subcore_axis_name="s", num_cores=NC, num_subcores=NS)


def _spmm_body(x_hbm, src_hbm, dst_hbm, w_hbm, zeros_hbm, p0, p1,
               acc_sh, src_v, dst_v, w_v, rows_v, gsem):
  c = lax.axis_index("c")
  s = lax.axis_index("s")
  wid = s * NC + c

  pltpu.sync_copy(zeros_hbm.at[pl.ds(s * ROWS_PT, ROWS_PT)],
                  acc_sh.at[pl.ds(s * ROWS_PT, ROWS_PT)])

  @pl.when(s == NS - 1)
  def _():
    pltpu.sync_copy(zeros_hbm.at[pl.ds(TAIL0, TAILR)],
                    acc_sh.at[pl.ds(TAIL0, TAILR)])
  pltpu.sync_copy(src_hbm.at[wid], src_v)
  pltpu.sync_copy(dst_hbm.at[wid], dst_v)
  pltpu.sync_copy(w_hbm.at[wid], w_v)
  plsc.subcore_barrier()

  def chunk_body(k, carry):
    pltpu.async_copy(x_hbm.at[src_v.at[k]], rows_v, gsem).wait()

    def group_body(g, _):
      w16 = w_v[k, pl.ds(g * L, L)]
      for lane in range(L):
        wv = w16[lane]
        e = g * L + lane
        for j in range(D // L):
          rows_v[e, pl.ds(j * L, L)] = rows_v[e, pl.ds(j * L, L)] * wv
      return 0

    lax.fori_loop(0, CH // L, group_body, 0)
    pltpu.sync_copy(rows_v, acc_sh.at[dst_v.at[k]], add=True)
    return carry

  lax.fori_loop(0, NCHK, chunk_body, 0)
  plsc.subcore_barrier()

  @pl.when(c == 0)
  def _():
    pltpu.sync_copy(acc_sh.at[pl.ds(s * ROWS_PT, ROWS_PT)],
                    p0.at[pl.ds(s * ROWS_PT, ROWS_PT)])

    @pl.when(s == NS - 1)
    def _():
      pltpu.sync_copy(acc_sh.at[pl.ds(TAIL0, TAILR)],
                      p0.at[pl.ds(TAIL0, TAILR)])

  @pl.when(c == 1)
  def _():
    pltpu.sync_copy(acc_sh.at[pl.ds(s * ROWS_PT, ROWS_PT)],
                    p1.at[pl.ds(s * ROWS_PT, ROWS_PT)])

    @pl.when(s == NS - 1)
    def _():
      pltpu.sync_copy(acc_sh.at[pl.ds(TAIL0, TAILR)],
                      p1.at[pl.ds(TAIL0, TAILR)])


_spmm = pl.kernel(
    _spmm_body,
    out_type=(jax.ShapeDtypeStruct((N, D), jnp.float32),
              jax.ShapeDtypeStruct((N, D), jnp.float32)),
    mesh=_MESH,
    scratch_types=[
        pltpu.VMEM_SHARED((N, D), jnp.float32),
        pltpu.VMEM((NCHK, CH), jnp.int32),
        pltpu.VMEM((NCHK, CH), jnp.int32),
        pltpu.VMEM((NCHK, CH), jnp.float32),
        pltpu.VMEM((CH, D), jnp.float32),
        pltpu.SemaphoreType.DMA,
    ],
)

NF = N * D
FPW = NF // NW
CF = 8000
NFC = FPW // CF


def _combine_body(scale, p0, p1, acc_in, x_out, acc_out, b0, b1, ba):
  c = lax.axis_index("c")
  s = lax.axis_index("s")
  base = (s * NC + c) * FPW

  def chunk(k, carry):
    off = base + k * CF
    pltpu.sync_copy(p0.at[pl.ds(off, CF)], b0)
    pltpu.sync_copy(p1.at[pl.ds(off, CF)], b1)
    pltpu.sync_copy(acc_in.at[pl.ds(off, CF)], ba)

    def vec(i, _):
      sl = pl.ds(i * L, L)
      x = b0[sl] + b1[sl]
      b0[sl] = x
      ba[sl] = (ba[sl] + x) * scale
      return 0

    lax.fori_loop(0, CF // L, vec, 0)
    pltpu.sync_copy(b0, x_out.at[pl.ds(off, CF)])
    pltpu.sync_copy(ba, acc_out.at[pl.ds(off, CF)])
    return carry

  lax.fori_loop(0, NFC, chunk, 0)


def _make_combine(scale):
  return pl.kernel(
      functools.partial(_combine_body, scale),
      out_type=(jax.ShapeDtypeStruct((NF,), jnp.float32),
                jax.ShapeDtypeStruct((NF,), jnp.float32)),
      mesh=_MESH,
      scratch_types=[
          pltpu.VMEM((CF,), jnp.float32),
          pltpu.VMEM((CF,), jnp.float32),
          pltpu.VMEM((CF,), jnp.float32),
      ],
  )


_combine_mid = _make_combine(1.0)
_combine_last = _make_combine(0.25)


def kernel(W, edge_weight, edge_index):
  dst = edge_index[0]
  src = edge_index[1]
  pad = EPW_P - EPW
  src3 = jnp.pad(src.reshape(NW, EPW), ((0, 0), (0, pad))).reshape(NW, NCHK, CH)
  dst3 = jnp.pad(dst.reshape(NW, EPW), ((0, 0), (0, pad))).reshape(NW, NCHK, CH)
  w3 = jnp.pad(edge_weight.reshape(NW, EPW),
               ((0, 0), (0, pad))).reshape(NW, NCHK, CH)
  zeros = jnp.zeros((N, D), jnp.float32)

  x = W
  acc = W.reshape(NF)
  for hop in range(3):
    p0, p1 = _spmm(x, src3, dst3, w3, zeros)
    comb = _combine_last if hop == 2 else _combine_mid
    xf, acc = comb(p0.reshape(NF), p1.reshape(NF), acc)
    x = xf.reshape(N, D)

  return acc.reshape(N, D), W
